# Initial kernel scaffold; baseline (speedup 1.0000x reference)
import jax
import jax.numpy as jnp
from jax.experimental import pallas as pl


def kernel(x, embedding):
    raise NotImplementedError("write your pallas kernel here")


# speedup vs baseline: 2.9675x; 2.9675x over previous
import functools

import jax
import jax.numpy as jnp
from jax import lax
from jax.experimental import pallas as pl
from jax.experimental.pallas import tpu as pltpu
from jax.experimental.pallas import tpu_sc as plsc

_NW = 32
_CH = 128


def kernel(x, embedding):
    B, S = x.shape
    V, D = embedding.shape
    total = B * S
    bpw = total // _NW
    nch = bpw // _CH
    assert bpw * _NW == total and nch * _CH == bpw

    idx = x.reshape(_NW, nch, _CH)
    mesh = plsc.VectorSubcoreMesh(core_axis_name="c", subcore_axis_name="s")

    @functools.partial(
        pl.kernel,
        out_type=jax.ShapeDtypeStruct((_NW, nch, _CH, D), jnp.float32),
        mesh=mesh,
        scratch_types=[
            pltpu.VMEM((nch, _CH), jnp.int32),
            pltpu.VMEM((_CH, D), jnp.float32),
            pltpu.SemaphoreType.DMA,
        ],
    )
    def emb_lookup(table_hbm, idx_hbm, out_hbm, idx_v, rows_v, gsem):
        wid = lax.axis_index("s") * 2 + lax.axis_index("c")
        pltpu.sync_copy(idx_hbm.at[wid], idx_v)

        @pl.loop(0, nch)
        def _chunk(j):
            pltpu.async_copy(table_hbm.at[idx_v.at[j]], rows_v, gsem).wait()
            pltpu.sync_copy(rows_v, out_hbm.at[wid, j])

    out = emb_lookup(embedding, idx)
    return out.reshape(B, S, D)


# speedup vs baseline: 3.3416x; 1.1261x over previous
import functools

import jax
import jax.numpy as jnp
from jax import lax
from jax.experimental import pallas as pl
from jax.experimental.pallas import tpu as pltpu
from jax.experimental.pallas import tpu_sc as plsc

_NW = 32
_CH = 128
_NBUF = 5


def kernel(x, embedding):
    B, S = x.shape
    V, D = embedding.shape
    total = B * S
    bpw = total // _NW
    nch = bpw // _CH
    assert bpw * _NW == total and nch * _CH == bpw and nch % _NBUF == 0

    idx = x.reshape(_NW, nch, _CH)
    mesh = plsc.VectorSubcoreMesh(core_axis_name="c", subcore_axis_name="s")

    @functools.partial(
        pl.kernel,
        out_type=jax.ShapeDtypeStruct((_NW, nch, _CH, D), jnp.float32),
        mesh=mesh,
        scratch_types=[
            pltpu.VMEM((nch, _CH), jnp.int32),
            pltpu.VMEM((_NBUF, _CH, D), jnp.float32),
        ]
        + [pltpu.SemaphoreType.DMA] * (2 * _NBUF),
    )
    def emb_lookup(table_hbm, idx_hbm, out_hbm, idx_v, rows_v, *sems):
        gsem, wsem = sems[:_NBUF], sems[_NBUF:]
        wid = lax.axis_index("s") * 2 + lax.axis_index("c")
        pltpu.sync_copy(idx_hbm.at[wid], idx_v)

        def start_gather(j, b):
            pltpu.async_copy(table_hbm.at[idx_v.at[j]], rows_v.at[b], gsem[b])

        for b in range(_NBUF):
            start_gather(b, b)

        @pl.loop(0, nch, step=_NBUF)
        def _group(j0):
            for b in range(_NBUF):
                j = j0 + b
                pltpu.make_async_copy(
                    table_hbm.at[idx_v.at[j]], rows_v.at[b], gsem[b]
                ).wait()
                pltpu.async_copy(rows_v.at[b], out_hbm.at[wid, j], wsem[b])

                @pl.when(j + _NBUF < nch)
                def _refill():
                    pltpu.make_async_copy(
                        rows_v.at[b], out_hbm.at[wid, j], wsem[b]
                    ).wait()
                    start_gather(j + _NBUF, b)

        for b in range(_NBUF):
            pltpu.make_async_copy(rows_v.at[b], out_hbm.at[wid, 0], wsem[b]).wait()

    out = emb_lookup(embedding, idx)
    return out.reshape(B, S, D)


# speedup vs baseline: 10.5077x; 3.1445x over previous
import functools

import jax
import jax.numpy as jnp
from jax import lax
from jax.experimental import pallas as pl
from jax.experimental.pallas import tpu as pltpu
from jax.experimental.pallas import tpu_sc as plsc

_NW = 32
_CH = 128
_NBUF = 5


def kernel(x, embedding):
    B, S = x.shape
    V, D = embedding.shape
    total = B * S
    bpw = total // _NW
    nch = bpw // _CH
    assert bpw * _NW == total and nch * _CH == bpw and nch % _NBUF == 0
    assert B % _CH == 0

    idx = x.T.reshape(_NW, nch, _CH)
    mesh = plsc.VectorSubcoreMesh(core_axis_name="c", subcore_axis_name="s")

    @functools.partial(
        pl.kernel,
        out_type=jax.ShapeDtypeStruct((S, B, D), jnp.float32),
        mesh=mesh,
        scratch_types=[
            pltpu.VMEM((nch, _CH), jnp.int32),
            pltpu.VMEM((_NBUF, _CH, D), jnp.float32),
        ]
        + [pltpu.SemaphoreType.DMA] * (2 * _NBUF),
    )
    def emb_lookup(table_hbm, idx_hbm, out_hbm, idx_v, rows_v, *sems):
        gsem, wsem = sems[:_NBUF], sems[_NBUF:]
        wid = lax.axis_index("s") * 2 + lax.axis_index("c")
        pltpu.sync_copy(idx_hbm.at[wid], idx_v)

        def start_gather(j, b):
            pltpu.async_copy(table_hbm.at[idx_v.at[j]], rows_v.at[b], gsem[b])

        def out_slice(j):
            flat = wid * bpw + j * _CH
            return out_hbm.at[flat // B, pl.ds(flat % B, _CH)]

        for b in range(_NBUF):
            start_gather(b, b)

        @pl.loop(0, nch, step=_NBUF)
        def _group(j0):
            for b in range(_NBUF):
                j = j0 + b
                pltpu.make_async_copy(
                    table_hbm.at[idx_v.at[j]], rows_v.at[b], gsem[b]
                ).wait()
                pltpu.async_copy(rows_v.at[b], out_slice(j), wsem[b])

                @pl.when(j + _NBUF < nch)
                def _refill():
                    pltpu.make_async_copy(
                        rows_v.at[b], out_slice(j), wsem[b]
                    ).wait()
                    start_gather(j + _NBUF, b)

        for b in range(_NBUF):
            pltpu.make_async_copy(rows_v.at[b], out_slice(0), wsem[b]).wait()

    out = emb_lookup(embedding, idx)
    return out.transpose(1, 0, 2)


# speedup vs baseline: 10.5326x; 1.0024x over previous
import functools

import jax
import jax.numpy as jnp
from jax import lax
from jax.experimental import pallas as pl
from jax.experimental.pallas import tpu as pltpu
from jax.experimental.pallas import tpu_sc as plsc

_NW = 32
_CH = 128
_NBUF = 7


def kernel(x, embedding):
    B, S = x.shape
    V, D = embedding.shape
    total = B * S
    bpw = total // _NW
    nch = bpw // _CH
    assert bpw * _NW == total and nch * _CH == bpw
    assert B % _CH == 0
    ngroups = nch // _NBUF

    idx = x.T.reshape(_NW, nch, _CH)
    mesh = plsc.VectorSubcoreMesh(core_axis_name="c", subcore_axis_name="s")

    @functools.partial(
        pl.kernel,
        out_type=jax.ShapeDtypeStruct((S, B, D), jnp.float32),
        mesh=mesh,
        scratch_types=[
            pltpu.VMEM((nch, _CH), jnp.int32),
            pltpu.VMEM((_NBUF, _CH, D), jnp.float32),
        ]
        + [pltpu.SemaphoreType.DMA] * (2 * _NBUF),
    )
    def emb_lookup(table_hbm, idx_hbm, out_hbm, idx_v, rows_v, *sems):
        gsem, wsem = sems[:_NBUF], sems[_NBUF:]
        wid = lax.axis_index("s") * 2 + lax.axis_index("c")
        pltpu.sync_copy(idx_hbm.at[wid], idx_v)

        def start_gather(j, b):
            pltpu.async_copy(table_hbm.at[idx_v.at[j]], rows_v.at[b], gsem[b])

        def out_slice(j):
            flat = wid * bpw + j * _CH
            return out_hbm.at[flat // B, pl.ds(flat % B, _CH)]

        for b in range(_NBUF):
            start_gather(b, b)

        @pl.loop(0, ngroups * _NBUF, step=_NBUF)
        def _group(j0):
            for b in range(_NBUF):
                j = j0 + b
                pltpu.make_async_copy(
                    table_hbm.at[idx_v.at[j]], rows_v.at[b], gsem[b]
                ).wait()
                pltpu.async_copy(rows_v.at[b], out_slice(j), wsem[b])

                @pl.when(j + _NBUF < nch)
                def _refill():
                    pltpu.make_async_copy(
                        rows_v.at[b], out_slice(j), wsem[b]
                    ).wait()
                    start_gather(j + _NBUF, b)

        for j in range(ngroups * _NBUF, nch):
            b = j % _NBUF
            pltpu.make_async_copy(
                table_hbm.at[idx_v.at[j]], rows_v.at[b], gsem[b]
            ).wait()
            pltpu.async_copy(rows_v.at[b], out_slice(j), wsem[b])

        for b in range(_NBUF):
            pltpu.make_async_copy(rows_v.at[b], out_slice(0), wsem[b]).wait()

    out = emb_lookup(embedding, idx)
    return out.transpose(1, 0, 2)
